# baseline (speedup 1.0000x reference)
import functools

import jax
import jax.numpy as jnp
from jax import lax
from jax.experimental import pallas as pl
from jax.experimental.pallas import tpu as pltpu
from jax.experimental.pallas import tpu_sc as plsc

B = 16384
LS = 20
LG = 7
NW = 32
CHUNK = B // NW
L = 16
NG = CHUNK // L
XD = 23


def _sc_pool(data_t, st_t, gt_t, gy_t):
    mesh = plsc.VectorSubcoreMesh(
        core_axis_name="c", subcore_axis_name="s", num_cores=2, num_subcores=16
    )

    @functools.partial(
        pl.kernel,
        out_type=jax.ShapeDtypeStruct((XD, B), jnp.float32),
        mesh=mesh,
        compiler_params=pltpu.CompilerParams(needs_layout_passes=False),
        scratch_types=[
            pltpu.VMEM((30, CHUNK), jnp.int32),
            pltpu.VMEM((10, 52), jnp.float32),
            pltpu.VMEM((4, 2), jnp.float32),
            pltpu.VMEM((4, 8), jnp.float32),
            pltpu.VMEM((XD, CHUNK), jnp.float32),
        ],
    )
    def k(data_hbm, st_hbm, gt_hbm, gy_hbm,
          out_hbm, d_v, st_v, gt_v, gy_v, x_v):
        wid = lax.axis_index("s") * 2 + lax.axis_index("c")
        base = pl.multiple_of(wid * CHUNK, CHUNK)
        pltpu.sync_copy(data_hbm.at[:, pl.ds(base, CHUNK)], d_v)
        pltpu.sync_copy(st_hbm, st_v)
        pltpu.sync_copy(gt_hbm, gt_v)
        pltpu.sync_copy(gy_hbm, gy_v)

        def body(g, carry):
            o = pl.multiple_of(g * L, L)
            acc_s = [jnp.zeros((L,), jnp.float32)] * 10
            for j in range(LS):
                idx = d_v[j, pl.ds(o, L)]
                for d in range(10):
                    acc_s[d] = acc_s[d] + plsc.load_gather(st_v.at[d], [idx])
            for d in range(10):
                x_v[d, pl.ds(o, L)] = acc_s[d] * (1.0 / LS)
            gi = d_v[27, pl.ds(o, L)]
            pi = d_v[28, pl.ds(o, L)]
            for d in range(4):
                x_v[10 + d, pl.ds(o, L)] = plsc.load_gather(gt_v.at[d], [gi])
                x_v[14 + d, pl.ds(o, L)] = plsc.load_gather(gt_v.at[d], [pi])
            acc_g = [jnp.zeros((L,), jnp.float32)] * 4
            for j in range(LG):
                idx = d_v[LS + j, pl.ds(o, L)]
                for d in range(4):
                    acc_g[d] = acc_g[d] + plsc.load_gather(gy_v.at[d], [idx])
            for d in range(4):
                x_v[18 + d, pl.ds(o, L)] = acc_g[d] * (1.0 / LG)
            age = plsc.bitcast(d_v[29, pl.ds(o, L)], jnp.float32)
            x_v[22, pl.ds(o, L)] = (age - 19.0) * (1.0 / 6.5)
            return carry

        lax.fori_loop(0, NG, body, None)
        pltpu.sync_copy(x_v, out_hbm.at[:, pl.ds(base, CHUNK)])

    return k(data_t, st_t, gt_t, gy_t)


def _tc_mlp(x_t, W1, b1, W2, b2):
    BB = 4096

    def body(x_ref, w1_ref, b1_ref, w2_ref, b2_ref, o_ref):
        x = x_ref[...]
        h = lax.dot_general(x, w1_ref[...], (((0,), (0,)), ((), ())),
                            preferred_element_type=jnp.float32)
        h = jnp.maximum(h + b1_ref[...], 0.0)
        o_ref[...] = jnp.dot(h, w2_ref[...],
                             preferred_element_type=jnp.float32) + b2_ref[...]

    return pl.pallas_call(
        body,
        grid=(B // BB,),
        in_specs=[
            pl.BlockSpec((XD, BB), lambda i: (0, i)),
            pl.BlockSpec((XD, 64), lambda i: (0, 0)),
            pl.BlockSpec((1, 64), lambda i: (0, 0)),
            pl.BlockSpec((64, 32), lambda i: (0, 0)),
            pl.BlockSpec((1, 32), lambda i: (0, 0)),
        ],
        out_specs=pl.BlockSpec((BB, 32), lambda i: (i, 0)),
        out_shape=jax.ShapeDtypeStruct((B, 32), jnp.float32),
    )(x_t, W1, b1.reshape(1, 64), W2, b2.reshape(1, 32))


def kernel(sports_ids, age, gender, preferred_gender, gym_days,
           sport_table, gender_table, gym_table, W1, b1, W2, b2):
    data_t = jnp.concatenate(
        [sports_ids.astype(jnp.int32), gym_days.astype(jnp.int32),
         gender.astype(jnp.int32), preferred_gender.astype(jnp.int32),
         lax.bitcast_convert_type(age, jnp.int32)],
        axis=1).T
    st_t = sport_table.T
    gt_t = gender_table.T
    gy_t = gym_table.T
    x_t = _sc_pool(data_t, st_t, gt_t, gy_t)
    return _tc_mlp(x_t, W1, b1, W2, b2)


# speedup vs baseline: 1.2940x; 1.2940x over previous
import functools

import jax
import jax.numpy as jnp
from jax import lax
from jax.experimental import pallas as pl
from jax.experimental.pallas import tpu as pltpu
from jax.experimental.pallas import tpu_sc as plsc

B = 16384
LS = 20
LG = 7
NW = 32
CHUNK = B // NW
L = 16
NG = CHUNK // L
XD = 23


def _sc_pool(ids_s_t, ids_g_t, g_ids, p_ids, age_v, st_t, gt_t, gy_t):
    mesh = plsc.VectorSubcoreMesh(
        core_axis_name="c", subcore_axis_name="s", num_cores=2, num_subcores=16
    )

    @functools.partial(
        pl.kernel,
        out_type=jax.ShapeDtypeStruct((XD, B), jnp.float32),
        mesh=mesh,
        compiler_params=pltpu.CompilerParams(needs_layout_passes=False),
        scratch_types=[
            pltpu.VMEM((LS, CHUNK), jnp.int32),
            pltpu.VMEM((LG, CHUNK), jnp.int32),
            pltpu.VMEM((CHUNK,), jnp.int32),
            pltpu.VMEM((CHUNK,), jnp.int32),
            pltpu.VMEM((CHUNK,), jnp.float32),
            pltpu.VMEM((5, 52), jnp.int32),
            pltpu.VMEM((2, 2), jnp.int32),
            pltpu.VMEM((2, 8), jnp.int32),
            pltpu.VMEM((XD, CHUNK), jnp.float32),
        ],
    )
    def k(ids_s_hbm, ids_g_hbm, g_hbm, p_hbm, a_hbm, st_hbm, gt_hbm, gy_hbm,
          out_hbm, ids_s_v, ids_g_v, g_v, p_v, a_v, st_v, gt_v, gy_v, x_v):
        wid = lax.axis_index("s") * 2 + lax.axis_index("c")
        base = pl.multiple_of(wid * CHUNK, CHUNK)
        pltpu.sync_copy(ids_s_hbm.at[:, pl.ds(base, CHUNK)], ids_s_v)
        pltpu.sync_copy(ids_g_hbm.at[:, pl.ds(base, CHUNK)], ids_g_v)
        pltpu.sync_copy(g_hbm.at[pl.ds(base, CHUNK)], g_v)
        pltpu.sync_copy(p_hbm.at[pl.ds(base, CHUNK)], p_v)
        pltpu.sync_copy(a_hbm.at[pl.ds(base, CHUNK)], a_v)
        pltpu.sync_copy(st_hbm, st_v)
        pltpu.sync_copy(gt_hbm, gt_v)
        pltpu.sync_copy(gy_hbm, gy_v)

        def unpk(w):
            return plsc.unpack(plsc.bitcast(w, jnp.bfloat16),
                               format=plsc.PackFormat.INTERLEAVED,
                               preferred_element_type=jnp.float32)

        def body(g, carry):
            o = pl.multiple_of(g * L, L)
            acc_s = [jnp.zeros((L,), jnp.float32)] * 10
            for j in range(LS):
                idx = ids_s_v[j, pl.ds(o, L)]
                for dp in range(5):
                    a, b = unpk(plsc.load_gather(st_v.at[dp], [idx]))
                    acc_s[2 * dp] = acc_s[2 * dp] + a
                    acc_s[2 * dp + 1] = acc_s[2 * dp + 1] + b
            for d in range(10):
                x_v[d, pl.ds(o, L)] = acc_s[d] * (1.0 / LS)
            gi = g_v[pl.ds(o, L)]
            pi = p_v[pl.ds(o, L)]
            for dp in range(2):
                a, b = unpk(plsc.load_gather(gt_v.at[dp], [gi]))
                x_v[10 + 2 * dp, pl.ds(o, L)] = a
                x_v[11 + 2 * dp, pl.ds(o, L)] = b
                a, b = unpk(plsc.load_gather(gt_v.at[dp], [pi]))
                x_v[14 + 2 * dp, pl.ds(o, L)] = a
                x_v[15 + 2 * dp, pl.ds(o, L)] = b
            acc_g = [jnp.zeros((L,), jnp.float32)] * 4
            for j in range(LG):
                idx = ids_g_v[j, pl.ds(o, L)]
                for dp in range(2):
                    a, b = unpk(plsc.load_gather(gy_v.at[dp], [idx]))
                    acc_g[2 * dp] = acc_g[2 * dp] + a
                    acc_g[2 * dp + 1] = acc_g[2 * dp + 1] + b
            for d in range(4):
                x_v[18 + d, pl.ds(o, L)] = acc_g[d] * (1.0 / LG)
            x_v[22, pl.ds(o, L)] = (a_v[pl.ds(o, L)] - 19.0) * (1.0 / 6.5)
            return carry

        lax.fori_loop(0, NG, body, None)
        pltpu.sync_copy(x_v, out_hbm.at[:, pl.ds(base, CHUNK)])

    return k(ids_s_t, ids_g_t, g_ids, p_ids, age_v, st_t, gt_t, gy_t)


def _tc_mlp(x_t, W1, b1, W2, b2):
    BB = 4096

    def body(x_ref, w1_ref, b1_ref, w2_ref, b2_ref, o_ref):
        x = x_ref[...]
        h = lax.dot_general(x, w1_ref[...], (((0,), (0,)), ((), ())),
                            preferred_element_type=jnp.float32)
        h = jnp.maximum(h + b1_ref[...], 0.0)
        o_ref[...] = jnp.dot(h, w2_ref[...],
                             preferred_element_type=jnp.float32) + b2_ref[...]

    return pl.pallas_call(
        body,
        grid=(B // BB,),
        in_specs=[
            pl.BlockSpec((XD, BB), lambda i: (0, i)),
            pl.BlockSpec((XD, 64), lambda i: (0, 0)),
            pl.BlockSpec((1, 64), lambda i: (0, 0)),
            pl.BlockSpec((64, 32), lambda i: (0, 0)),
            pl.BlockSpec((1, 32), lambda i: (0, 0)),
        ],
        out_specs=pl.BlockSpec((BB, 32), lambda i: (i, 0)),
        out_shape=jax.ShapeDtypeStruct((B, 32), jnp.float32),
    )(x_t, W1, b1.reshape(1, 64), W2, b2.reshape(1, 32))


def _pack_pairs(t):
    tb = t.astype(jnp.bfloat16)
    lo = lax.bitcast_convert_type(tb[:, 0::2], jnp.uint16).astype(jnp.uint32)
    hi = lax.bitcast_convert_type(tb[:, 1::2], jnp.uint16).astype(jnp.uint32)
    return lax.bitcast_convert_type(lo | (hi << 16), jnp.int32).T


def kernel(sports_ids, age, gender, preferred_gender, gym_days,
           sport_table, gender_table, gym_table, W1, b1, W2, b2):
    ids_s_t = sports_ids.T.astype(jnp.int32)
    ids_g_t = gym_days.T.astype(jnp.int32)
    g_ids = gender[:, 0].astype(jnp.int32)
    p_ids = preferred_gender[:, 0].astype(jnp.int32)
    age_v = age[:, 0]
    stp = _pack_pairs(sport_table)
    gtp = _pack_pairs(gender_table)
    gyp = _pack_pairs(gym_table)
    x_t = _sc_pool(ids_s_t, ids_g_t, g_ids, p_ids, age_v, stp, gtp, gyp)
    return _tc_mlp(x_t, W1, b1, W2, b2)
